# Initial kernel scaffold; baseline (speedup 1.0000x reference)
import jax
import jax.numpy as jnp
from jax.experimental import pallas as pl


def kernel(input_ids, word_table, pos_table, gamma, beta):
    raise NotImplementedError("write your pallas kernel here")


# speedup vs baseline: 1.3038x; 1.3038x over previous
import functools

import jax
import jax.numpy as jnp
from jax import lax
from jax.experimental import pallas as pl
from jax.experimental.pallas import tpu as pltpu
from jax.experimental.pallas import tpu_sc as plsc

EPS = 1e-12

NUM_WORKERS = 32
GATHER_CHUNK = 64


def _sc_gather(table, idx):
    b, = idx.shape
    _, d = table.shape
    b_per_w = b // NUM_WORKERS
    mesh = plsc.VectorSubcoreMesh(core_axis_name="c", subcore_axis_name="s")

    @functools.partial(
        pl.kernel,
        mesh=mesh,
        out_type=jax.ShapeDtypeStruct((b, d), jnp.float32),
        scratch_types=[
            pltpu.VMEM((b_per_w,), jnp.int32),
            pltpu.VMEM((GATHER_CHUNK, d), jnp.float32),
            pltpu.SemaphoreType.DMA,
        ],
    )
    def gather_kernel(table_hbm, idx_hbm, out_hbm, idx_v, rows_v, sem):
        wid = lax.axis_index("s") * 2 + lax.axis_index("c")
        base = wid * b_per_w
        pltpu.sync_copy(idx_hbm.at[pl.ds(base, b_per_w)], idx_v)

        @pl.loop(0, b_per_w, step=GATHER_CHUNK)
        def _(c):
            pltpu.async_copy(
                table_hbm.at[idx_v.at[pl.ds(c, GATHER_CHUNK)]], rows_v, sem
            ).wait()
            pltpu.sync_copy(rows_v, out_hbm.at[pl.ds(base + c, GATHER_CHUNK)])

    return gather_kernel(table, idx)


def _ln_body(x_ref, pos_ref, gamma_ref, beta_ref, out_ref):
    x = x_ref[...] + pos_ref[...]
    mean = jnp.mean(x, axis=-1, keepdims=True)
    centered = x - mean
    var = jnp.mean(centered * centered, axis=-1, keepdims=True)
    normed = centered * lax.rsqrt(var + EPS)
    out_ref[...] = normed * gamma_ref[...] + beta_ref[...]


def _tc_add_ln(gathered, pos_table, gamma, beta, block_rows):
    n, d = gathered.shape
    s = pos_table.shape[0]
    pos_blocks = s // block_rows
    grid = (n // block_rows,)
    return pl.pallas_call(
        _ln_body,
        grid=grid,
        in_specs=[
            pl.BlockSpec((block_rows, d), lambda i: (i, 0)),
            pl.BlockSpec((block_rows, d), lambda i: (i % pos_blocks, 0)),
            pl.BlockSpec((1, d), lambda i: (0, 0)),
            pl.BlockSpec((1, d), lambda i: (0, 0)),
        ],
        out_specs=pl.BlockSpec((block_rows, d), lambda i: (i, 0)),
        out_shape=jax.ShapeDtypeStruct((n, d), jnp.float32),
        compiler_params=pltpu.CompilerParams(
            dimension_semantics=("parallel",),
        ),
    )(gathered, pos_table, gamma.reshape(1, d), beta.reshape(1, d))


def kernel(input_ids, word_table, pos_table, gamma, beta):
    batch, seq = input_ids.shape
    d = word_table.shape[1]
    ids_flat = input_ids.reshape(-1).astype(jnp.int32)
    gathered = _sc_gather(word_table, ids_flat)
    out = _tc_add_ln(gathered, pos_table, gamma, beta, block_rows=512)
    return out.reshape(batch, seq, d)


# speedup vs baseline: 1.4116x; 1.0827x over previous
import functools

import jax
import jax.numpy as jnp
from jax import lax
from jax.experimental import pallas as pl
from jax.experimental.pallas import tpu as pltpu
from jax.experimental.pallas import tpu_sc as plsc

EPS = 1e-12

NUM_WORKERS = 32
GATHER_CHUNK = 64


def _sc_gather(table, idx):
    b, = idx.shape
    _, d = table.shape
    b_per_w = b // NUM_WORKERS
    mesh = plsc.VectorSubcoreMesh(core_axis_name="c", subcore_axis_name="s")

    @functools.partial(
        pl.kernel,
        mesh=mesh,
        out_type=jax.ShapeDtypeStruct((b, d), jnp.float32),
        scratch_types=[
            pltpu.VMEM((b_per_w,), jnp.int32),
            pltpu.VMEM((GATHER_CHUNK, d), jnp.float32),
            pltpu.SemaphoreType.DMA,
        ],
    )
    def gather_kernel(table_hbm, idx_hbm, out_hbm, idx_v, rows_v, sem):
        wid = lax.axis_index("s") * 2 + lax.axis_index("c")
        base = wid * b_per_w
        pltpu.sync_copy(idx_hbm.at[pl.ds(base, b_per_w)], idx_v)

        @pl.loop(0, b_per_w, step=GATHER_CHUNK)
        def _(c):
            pltpu.async_copy(
                table_hbm.at[idx_v.at[pl.ds(c, GATHER_CHUNK)]], rows_v, sem
            ).wait()
            pltpu.sync_copy(rows_v, out_hbm.at[pl.ds(base + c, GATHER_CHUNK)])

    return gather_kernel(table, idx)


def _ln_body(block_rows, pos_period, x_ref, pos_ref, gamma_ref, beta_ref, out_ref):
    pos_start = (pl.program_id(0) % pos_period) * block_rows
    x = x_ref[...] + pos_ref[pl.ds(pos_start, block_rows), :]
    mean = jnp.mean(x, axis=-1, keepdims=True)
    centered = x - mean
    var = jnp.mean(centered * centered, axis=-1, keepdims=True)
    normed = centered * lax.rsqrt(var + EPS)
    out_ref[...] = normed * gamma_ref[...] + beta_ref[...]


def _tc_add_ln(gathered, pos_table, gamma, beta, block_rows):
    n, d = gathered.shape
    s = pos_table.shape[0]
    pos_period = s // block_rows
    grid = (n // block_rows,)
    return pl.pallas_call(
        functools.partial(_ln_body, block_rows, pos_period),
        grid=grid,
        in_specs=[
            pl.BlockSpec((block_rows, d), lambda i: (i, 0)),
            pl.BlockSpec((s, d), lambda i: (0, 0)),
            pl.BlockSpec((1, d), lambda i: (0, 0)),
            pl.BlockSpec((1, d), lambda i: (0, 0)),
        ],
        out_specs=pl.BlockSpec((block_rows, d), lambda i: (i, 0)),
        out_shape=jax.ShapeDtypeStruct((n, d), jnp.float32),
        compiler_params=pltpu.CompilerParams(
            dimension_semantics=("arbitrary",),
        ),
    )(gathered, pos_table, gamma.reshape(1, d), beta.reshape(1, d))


def kernel(input_ids, word_table, pos_table, gamma, beta):
    batch, seq = input_ids.shape
    d = word_table.shape[1]
    ids_flat = input_ids.reshape(-1).astype(jnp.int32)
    gathered = _sc_gather(word_table, ids_flat)
    out = _tc_add_ln(gathered, pos_table, gamma, beta, block_rows=1024)
    return out.reshape(batch, seq, d)
